# baseline (speedup 1.0000x reference)
import functools

import jax
import jax.numpy as jnp
from jax import lax
from jax.experimental import pallas as pl
from jax.experimental.pallas import tpu as pltpu
from jax.experimental.pallas import tpu_sc as plsc

B = 16384
D = 32
NC = 2
NS = 16
NW = NC * NS
BPW = B // NW
HALF = BPW // 2
TROWS = 250000

_MESH = plsc.VectorSubcoreMesh(core_axis_name="c", subcore_axis_name="s")


@functools.partial(
    pl.kernel,
    mesh=_MESH,
    out_type=[jax.ShapeDtypeStruct((B, 128), jnp.float32)] * 4,
    scratch_types=[
        pltpu.VMEM((BPW,), jnp.int32),
        pltpu.VMEM((BPW,), jnp.int32),
        pltpu.VMEM((HALF, 128), jnp.float32),
        pltpu.VMEM((HALF, 128), jnp.float32),
        pltpu.SemaphoreType.DMA,
        pltpu.SemaphoreType.DMA,
    ],
)
def _sc_gather(qu_hbm, qi_hbm, ug_hbm, ig_hbm, um_hbm, im_hbm,
               gu_out, gi_out, hu_out, hi_out,
               qu_v, qi_v, buf0, buf1, sem0, sem1):
    wid = lax.axis_index("s") * NC + lax.axis_index("c")
    base = wid * BPW
    pltpu.sync_copy(qu_hbm.at[pl.ds(base, BPW)], qu_v)
    pltpu.sync_copy(qi_hbm.at[pl.ds(base, BPW)], qi_v)

    bufs = (buf0, buf1)
    sems = (sem0, sem1)
    rounds = [
        (ug_hbm, qu_v, 0, gu_out), (ug_hbm, qu_v, 1, gu_out),
        (ig_hbm, qi_v, 0, gi_out), (ig_hbm, qi_v, 1, gi_out),
        (um_hbm, qu_v, 0, hu_out), (um_hbm, qu_v, 1, hu_out),
        (im_hbm, qi_v, 0, hi_out), (im_hbm, qi_v, 1, hi_out),
    ]

    def fire(k):
        tbl, qv, h, _ = rounds[k]
        return pltpu.async_copy(
            tbl.at[qv.at[pl.ds(h * HALF, HALF)]], bufs[k % 2], sems[k % 2])

    cps = [None] * 8
    cps[0] = fire(0)
    for k in range(8):
        if k + 1 < 8:
            cps[k + 1] = fire(k + 1)
        cps[k].wait()
        _, _, h, out = rounds[k]
        pltpu.sync_copy(bufs[k % 2], out.at[pl.ds(base + h * HALF, HALF)])


BLK = 2048


def _sel(big, s):
    out = big[:, 0:D]
    for k in range(1, 4):
        out = jnp.where(s == k, big[:, k * D:(k + 1) * D], out)
    return out


def _mlp_body(gu, gi, hu, hi, su, si, w1u, w1i, b1, w2, b2, w3, b3, wfg, wfh,
              bf, out):
    su_ = su[...]
    si_ = si[...]
    g = _sel(gu[...], su_) * _sel(gi[...], si_)
    h = jnp.dot(_sel(hu[...], su_), w1u[...],
                preferred_element_type=jnp.float32)
    h = h + jnp.dot(_sel(hi[...], si_), w1i[...],
                    preferred_element_type=jnp.float32)
    h = jnp.maximum(h + b1[...], 0.0)
    h = jnp.maximum(
        jnp.dot(h, w2[...], preferred_element_type=jnp.float32) + b2[...], 0.0)
    h = jnp.maximum(
        jnp.dot(h, w3[...], preferred_element_type=jnp.float32) + b3[...], 0.0)
    out[...] = (jnp.dot(g, wfg[...], preferred_element_type=jnp.float32)
                + jnp.dot(h, wfh[...], preferred_element_type=jnp.float32)
                + bf[...])


def _mlp(gu, gi, hu, hi, su, si, W1u, W1i, b1, W2, b2, W3, b3, Wfg, Wfh, bf):
    grid = (B // BLK,)
    big_spec = pl.BlockSpec((BLK, 128), lambda j: (j, 0))
    s_spec = pl.BlockSpec((BLK, 1), lambda j: (j, 0))
    full = lambda s: pl.BlockSpec(s, lambda j: (0,) * len(s))
    return pl.pallas_call(
        _mlp_body,
        grid=grid,
        in_specs=[
            big_spec, big_spec, big_spec, big_spec, s_spec, s_spec,
            full((D, 64)), full((D, 64)), full((1, 64)),
            full((64, 32)), full((1, 32)),
            full((32, 16)), full((1, 16)),
            full((D, 1)), full((16, 1)), full((1, 1)),
        ],
        out_specs=pl.BlockSpec((BLK, 1), lambda j: (j, 0)),
        out_shape=jax.ShapeDtypeStruct((B, 1), jnp.float32),
    )(gu, gi, hu, hi, su, si, W1u, W1i, b1, W2, b2, W3, b3, Wfg, Wfh, bf)


def kernel(u, i, ug, ig, um, im, W1, b1, W2, b2, W3, b3, Wf, bf):
    u = u.astype(jnp.int32)
    i = i.astype(jnp.int32)
    qu, su = u >> 2, (u & 3).reshape(B, 1)
    qi, si = i >> 2, (i & 3).reshape(B, 1)
    ugr = ug.reshape(TROWS, 128)
    igr = ig.reshape(TROWS, 128)
    umr = um.reshape(TROWS, 128)
    imr = im.reshape(TROWS, 128)
    gu, gi, hu, hi = _sc_gather(qu, qi, ugr, igr, umr, imr)
    out = _mlp(gu, gi, hu, hi, su, si,
               W1[:D], W1[D:], b1.reshape(1, 64),
               W2, b2.reshape(1, 32), W3, b3.reshape(1, 16),
               Wf[:D], Wf[D:], bf.reshape(1, 1))
    return out.reshape(-1)


# speedup vs baseline: 1.4491x; 1.4491x over previous
import functools

import jax
import jax.numpy as jnp
from jax import lax
from jax.experimental import pallas as pl
from jax.experimental.pallas import tpu as pltpu
from jax.experimental.pallas import tpu_sc as plsc

B = 16384
D = 32
NC = 2
NS = 16
NW = NC * NS
BPW = B // NW

_MESH = plsc.VectorSubcoreMesh(core_axis_name="c", subcore_axis_name="s")


@functools.partial(
    pl.kernel,
    mesh=_MESH,
    out_type=jax.ShapeDtypeStruct((B, D), jnp.float32),
    scratch_types=[
        pltpu.SMEM((BPW,), jnp.int32),
        pltpu.VMEM((BPW,), jnp.int32),
        pltpu.VMEM_SHARED((NS, BPW), jnp.int32),
        pltpu.VMEM((BPW, D), jnp.float32),
        pltpu.SemaphoreType.DMA,
    ],
)
def _sc_gather(idx_hbm, tbl, out, idx_sm, idx_v, idx_sp, buf, sem):
    sid = lax.axis_index("s")
    wid = sid * NC + lax.axis_index("c")
    base = wid * BPW
    pltpu.sync_copy(idx_hbm.at[pl.ds(base, BPW)], idx_v)
    pltpu.sync_copy(idx_v, idx_sp.at[sid])
    pltpu.sync_copy(idx_sp.at[sid], idx_sm)

    def body(j, _):
        r = idx_sm[j]
        pltpu.async_copy(tbl.at[pl.ds(r, 1), :], buf.at[pl.ds(j, 1), :], sem)
        return ()

    lax.fori_loop(0, BPW, body, (), unroll=2)
    pltpu.make_async_copy(tbl.at[pl.ds(0, BPW), :], buf, sem).wait()
    pltpu.sync_copy(buf, out.at[pl.ds(base, BPW)])


BLK = 2048


def _mlp_body(gu, gi, hu, hi, w1u, w1i, b1, w2, b2, w3, b3, wfg, wfh, bf,
              out):
    g = gu[...] * gi[...]
    h = jnp.dot(hu[...], w1u[...], preferred_element_type=jnp.float32)
    h = h + jnp.dot(hi[...], w1i[...], preferred_element_type=jnp.float32)
    h = jnp.maximum(h + b1[...], 0.0)
    h = jnp.maximum(
        jnp.dot(h, w2[...], preferred_element_type=jnp.float32) + b2[...], 0.0)
    h = jnp.maximum(
        jnp.dot(h, w3[...], preferred_element_type=jnp.float32) + b3[...], 0.0)
    out[...] = (jnp.dot(g, wfg[...], preferred_element_type=jnp.float32)
                + jnp.dot(h, wfh[...], preferred_element_type=jnp.float32)
                + bf[...])


def _mlp(gu, gi, hu, hi, W1u, W1i, b1, W2, b2, W3, b3, Wfg, Wfh, bf):
    grid = (B // BLK,)
    row_spec = pl.BlockSpec((BLK, D), lambda j: (j, 0))
    full = lambda s: pl.BlockSpec(s, lambda j: (0,) * len(s))
    return pl.pallas_call(
        _mlp_body,
        grid=grid,
        in_specs=[
            row_spec, row_spec, row_spec, row_spec,
            full((D, 64)), full((D, 64)), full((1, 64)),
            full((64, 32)), full((1, 32)),
            full((32, 16)), full((1, 16)),
            full((D, 1)), full((16, 1)), full((1, 1)),
        ],
        out_specs=pl.BlockSpec((BLK, 1), lambda j: (j, 0)),
        out_shape=jax.ShapeDtypeStruct((B, 1), jnp.float32),
    )(gu, gi, hu, hi, W1u, W1i, b1, W2, b2, W3, b3, Wfg, Wfh, bf)


def kernel(u, i, ug, ig, um, im, W1, b1, W2, b2, W3, b3, Wf, bf):
    u = u.astype(jnp.int32)
    i = i.astype(jnp.int32)
    gu = _sc_gather(u, ug)
    gi = _sc_gather(i, ig)
    hu = _sc_gather(u, um)
    hi = _sc_gather(i, im)
    out = _mlp(gu, gi, hu, hi,
               W1[:D], W1[D:], b1.reshape(1, 64),
               W2, b2.reshape(1, 32), W3, b3.reshape(1, 16),
               Wf[:D], Wf[D:], bf.reshape(1, 1))
    return out.reshape(-1)
